# Initial kernel scaffold; baseline (speedup 1.0000x reference)
import jax
import jax.numpy as jnp
from jax.experimental import pallas as pl


def kernel(sequence, table, W, b, gamma, beta):
    raise NotImplementedError("write your pallas kernel here")


# speedup vs baseline: 1.6556x; 1.6556x over previous
import functools
import numpy as np
import jax
import jax.numpy as jnp
from jax import lax
from jax.experimental import pallas as pl
from jax.experimental.pallas import tpu as pltpu
from jax.experimental.pallas import tpu_sc as plsc

_EPS = 1e-5
_MAXLEN = 512

_NW = 32
_C = 512
_KBUF = 2


def _pe_table(S, D):
    pos = np.arange(_MAXLEN, dtype=np.float32)[:, None]
    i = np.arange(0, D, 2, dtype=np.float32)
    div = np.exp(-(np.log(10000.0)) * i / D)
    pe = np.zeros((_MAXLEN, D), dtype=np.float32)
    pe[:, 0::2] = np.sin(pos * div)
    pe[:, 1::2] = np.cos(pos * div)
    return jnp.asarray(pe[:S])


def _make_sc_gather(n_w, n_chunks, E):
    mesh = plsc.VectorSubcoreMesh(core_axis_name="c", subcore_axis_name="s")

    idx_scratch = [pltpu.VMEM((_C,), jnp.int32) for _ in range(_KBUF)]
    row_scratch = [pltpu.VMEM((_C, E), jnp.float32) for _ in range(_KBUF)]

    @functools.partial(
        pl.kernel,
        mesh=mesh,
        out_type=jax.ShapeDtypeStruct((_NW, n_w, E), jnp.float32),
        scratch_types=idx_scratch + row_scratch + [pltpu.SemaphoreType.DMA],
        compiler_params=pltpu.CompilerParams(use_tc_tiling_on_sc=False),
    )
    def sc_gather(seq_hbm, table_hbm, out_hbm, *scratch):
        idx_v = scratch[:_KBUF]
        rows_v = scratch[_KBUF : 2 * _KBUF]
        sem = scratch[2 * _KBUF]
        wid = lax.axis_index("s") * 2 + lax.axis_index("c")

        def group(g, _):
            base = g * (_KBUF * _C)
            copies = []
            for bf in range(_KBUF):
                pltpu.sync_copy(seq_hbm.at[wid, g * _KBUF + bf], idx_v[bf])
                cp = pltpu.async_copy(
                    table_hbm.at[idx_v[bf]], rows_v[bf], sem
                )
                copies.append(cp)
            for bf in range(_KBUF):
                copies[bf].wait()
                pltpu.sync_copy(
                    rows_v[bf], out_hbm.at[wid, pl.ds(base + bf * _C, _C)]
                )
            return _

        lax.fori_loop(0, n_chunks // _KBUF, group, None)

    return sc_gather


def _tc_body(BB, seq_ref, emb_ref, wt_ref, pb_ref, g_ref, bt_ref, out_ref):
    wt = wt_ref[...]
    pb = pb_ref[...]
    gm = g_ref[...]
    bt = bt_ref[...]
    for i in range(BB):
        m = (seq_ref[i] != 0).astype(jnp.float32)
        e = emb_ref[i] * m
        x = jnp.dot(e, wt, preferred_element_type=jnp.float32) + pb
        mu = jnp.mean(x, axis=-1, keepdims=True)
        xc = x - mu
        var = jnp.mean(xc * xc, axis=-1, keepdims=True)
        out_ref[i] = (xc * lax.rsqrt(var + _EPS)) * gm + bt


def kernel(sequence, table, W, b, gamma, beta):
    B, S = sequence.shape
    V, E = table.shape
    D = W.shape[0]
    N = B * S
    assert N % (_NW * _C) == 0
    n_w = N // _NW
    n_chunks = n_w // _C

    seq_i32 = sequence.astype(jnp.int32)
    seq_flat = seq_i32.reshape(_NW, n_chunks, _C)
    emb = _make_sc_gather(n_w, n_chunks, E)(seq_flat, table)
    emb = emb.reshape(B, S, E)

    pb = _pe_table(S, D) + b[None, :]
    wt = W.T

    BB = 8
    grid = (B // BB,)
    out = pl.pallas_call(
        functools.partial(_tc_body, BB),
        grid=grid,
        in_specs=[
            pl.BlockSpec((BB, S, 1), lambda i: (i, 0, 0)),
            pl.BlockSpec((BB, S, E), lambda i: (i, 0, 0)),
            pl.BlockSpec((E, D), lambda i: (0, 0)),
            pl.BlockSpec((S, D), lambda i: (0, 0)),
            pl.BlockSpec((1, D), lambda i: (0, 0)),
            pl.BlockSpec((1, D), lambda i: (0, 0)),
        ],
        out_specs=pl.BlockSpec((BB, S, D), lambda i: (i, 0, 0)),
        out_shape=jax.ShapeDtypeStruct((B, S, D), jnp.float32),
        compiler_params=pltpu.CompilerParams(
            dimension_semantics=("arbitrary",),
        ),
    )(
        seq_i32.reshape(B, S, 1),
        emb,
        wt,
        pb,
        gamma.reshape(1, D),
        beta.reshape(1, D),
    )
    return out


# speedup vs baseline: 1.9074x; 1.1520x over previous
import functools
import numpy as np
import jax
import jax.numpy as jnp
from jax import lax
from jax.experimental import pallas as pl
from jax.experimental.pallas import tpu as pltpu
from jax.experimental.pallas import tpu_sc as plsc

_EPS = 1e-5
_MAXLEN = 512

_NW = 32
_C = 512
_KBUF = 2


def _pe_table(S, D):
    pos = np.arange(_MAXLEN, dtype=np.float32)[:, None]
    i = np.arange(0, D, 2, dtype=np.float32)
    div = np.exp(-(np.log(10000.0)) * i / D)
    pe = np.zeros((_MAXLEN, D), dtype=np.float32)
    pe[:, 0::2] = np.sin(pos * div)
    pe[:, 1::2] = np.cos(pos * div)
    return jnp.asarray(pe[:S])


def _make_sc_gather(n_w, n_chunks, E):
    mesh = plsc.VectorSubcoreMesh(core_axis_name="c", subcore_axis_name="s")
    idx_scratch = [pltpu.VMEM((_C,), jnp.int32) for _ in range(_KBUF)]
    row_scratch = [pltpu.VMEM((_C, E), jnp.float32) for _ in range(_KBUF)]

    @functools.partial(
        pl.kernel,
        mesh=mesh,
        out_type=jax.ShapeDtypeStruct((_NW, n_w, E), jnp.float32),
        scratch_types=idx_scratch + row_scratch + [pltpu.SemaphoreType.DMA],
        compiler_params=pltpu.CompilerParams(use_tc_tiling_on_sc=False),
    )
    def sc_gather(seq_hbm, table_hbm, out_hbm, *scratch):
        idx_v = scratch[:_KBUF]
        rows_v = scratch[_KBUF : 2 * _KBUF]
        sem = scratch[2 * _KBUF]
        wid = lax.axis_index("s") * 2 + lax.axis_index("c")

        def group(g, _):
            base = g * (_KBUF * _C)
            copies = []
            for bf in range(_KBUF):
                pltpu.sync_copy(seq_hbm.at[wid, g * _KBUF + bf], idx_v[bf])
                cp = pltpu.async_copy(
                    table_hbm.at[idx_v[bf]], rows_v[bf], sem
                )
                copies.append(cp)
            for bf in range(_KBUF):
                copies[bf].wait()
                pltpu.sync_copy(
                    rows_v[bf], out_hbm.at[wid, pl.ds(base + bf * _C, _C)]
                )
            return _

        lax.fori_loop(0, n_chunks // _KBUF, group, None)

    return sc_gather


def _tc_body(BB, HS, emb_ref, wlo_ref, whi_ref, pb_ref, g_ref,
             bt_ref, out_ref):
    wlo = wlo_ref[...]
    whi = whi_ref[...]
    pb = pb_ref[...]
    gm = g_ref[...]
    bt = bt_ref[...]

    def ln(x):
        mu = jnp.mean(x, axis=-1, keepdims=True)
        xc = x - mu
        var = jnp.mean(xc * xc, axis=-1, keepdims=True)
        return (xc * lax.rsqrt(var + _EPS)) * gm + bt

    for i in range(BB):
        p = emb_ref[i * HS : (i + 1) * HS, :]
        x_lo = jnp.dot(p, wlo, preferred_element_type=jnp.float32)
        out_ref[i, 0:HS] = ln(x_lo + pb[0:HS])
        x_hi = jnp.dot(p, whi, preferred_element_type=jnp.float32)
        out_ref[i, HS : 2 * HS] = ln(x_hi + pb[HS : 2 * HS])


def kernel(sequence, table, W, b, gamma, beta):
    B, S = sequence.shape
    V, E = table.shape
    D = W.shape[0]
    N = B * S
    HS = S // 2
    assert S % 2 == 0 and N % (_NW * _C) == 0
    n_w = N // _NW
    n_chunks = n_w // _C

    table_ext = jnp.concatenate(
        [table, jnp.zeros((8, E), jnp.float32)], axis=0
    )
    seq_i32 = sequence.astype(jnp.int32)
    seq_remap = jnp.where(seq_i32 == 0, V, seq_i32)
    seq_perm = jnp.stack(
        [seq_remap[:, :HS], seq_remap[:, HS:]], axis=-1
    )
    seq_flat = seq_perm.reshape(_NW, n_chunks, _C)

    emb = _make_sc_gather(n_w, n_chunks, E)(seq_flat, table_ext)
    emb2 = emb.reshape(N // 2, 2 * E)

    pb = _pe_table(S, D) + b[None, :]
    wlo = jnp.concatenate([W.T, jnp.zeros((E, D), jnp.float32)], axis=0)
    whi = jnp.concatenate([jnp.zeros((E, D), jnp.float32), W.T], axis=0)

    BB = 16
    grid = (B // BB,)
    out = pl.pallas_call(
        functools.partial(_tc_body, BB, HS),
        grid=grid,
        in_specs=[
            pl.BlockSpec((BB * HS, 2 * E), lambda i: (i, 0)),
            pl.BlockSpec((2 * E, D), lambda i: (0, 0)),
            pl.BlockSpec((2 * E, D), lambda i: (0, 0)),
            pl.BlockSpec((S, D), lambda i: (0, 0)),
            pl.BlockSpec((1, D), lambda i: (0, 0)),
            pl.BlockSpec((1, D), lambda i: (0, 0)),
        ],
        out_specs=pl.BlockSpec((BB, S, D), lambda i: (i, 0, 0)),
        out_shape=jax.ShapeDtypeStruct((B, S, D), jnp.float32),
        compiler_params=pltpu.CompilerParams(
            dimension_semantics=("arbitrary",),
        ),
    )(
        emb2,
        wlo,
        whi,
        pb,
        gamma.reshape(1, D),
        beta.reshape(1, D),
    )
    return out
